# Initial kernel scaffold; baseline (speedup 1.0000x reference)
import jax
import jax.numpy as jnp
from jax.experimental import pallas as pl


def kernel(b, q, edge_index, edge_attr, norm_v2c, norm_c2v, batch_vals, batch_cons, num_graphs, be_W1, be_b1, be_W2, be_b2, qe_W1, qe_b1, qe_W2, qe_b2, conv0_v2c_W1, conv0_v2c_b1, conv0_v2c_W2, conv0_v2c_b2, conv0_c2v_W1, conv0_c2v_b1, conv0_c2v_W2, conv0_c2v_b2, conv1_v2c_W1, conv1_v2c_b1, conv1_v2c_W2, conv1_v2c_b2, conv1_c2v_W1, conv1_c2v_b1, conv1_c2v_W2, conv1_c2v_b2, conv2_v2c_W1, conv2_v2c_b1, conv2_v2c_W2, conv2_v2c_b2, conv2_c2v_W1, conv2_c2v_b1, conv2_c2v_W2, conv2_c2v_b2, fc_W, fc_b):
    raise NotImplementedError("write your pallas kernel here")


# speedup vs baseline: 3.8108x; 3.8108x over previous
import functools

import jax
import jax.numpy as jnp
from jax import lax
from jax.experimental import pallas as pl
from jax.experimental.pallas import tpu as pltpu
from jax.experimental.pallas import tpu_sc as plsc

N = 5000
E = 320000
H = 128
NG = 16
NC = 2
NS = 16
NW = NC * NS
EPW = E // NW
K = 80
NCHUNK = EPW // K
NP = 5120
RPS = NP // NS
ZR = 64


def _sc_agg_body(nodes, srcs, dsts, cfs, out, idx_v, dst_v, cf_v,
                 rows_v, zb_v, sem, acc):
    c = lax.axis_index("c")
    s = lax.axis_index("s")
    w = c * NS + s

    z16 = jnp.zeros((16,), jnp.float32)

    def zrow(i, _):
        for j in range(H // 16):
            zb_v[i, pl.ds(j * 16, 16)] = z16
        return 0

    lax.fori_loop(0, ZR, zrow, 0)
    r0 = s * RPS
    for t in range(RPS // ZR):
        pltpu.sync_copy(zb_v, acc.at[pl.ds(r0 + t * ZR, ZR), :])
    plsc.subcore_barrier()

    base_w = w * EPW

    def chunk(g, _):
        base = base_w + g * K
        pltpu.sync_copy(srcs.at[pl.ds(base, K)], idx_v)
        pltpu.sync_copy(dsts.at[pl.ds(base, K)], dst_v)
        pltpu.sync_copy(cfs.at[pl.ds(base, K)], cf_v)
        pltpu.async_copy(nodes.at[idx_v], rows_v, sem).wait()

        def group(g2, _):
            cfg = cf_v[pl.ds(g2 * 16, 16)]
            for i in range(16):
                cfb = cfg.at[jnp.full((16,), i, jnp.int32)].get(
                    mode="promise_in_bounds")
                r = g2 * 16 + i
                for j in range(H // 16):
                    sl = pl.ds(j * 16, 16)
                    rows_v[r, sl] = rows_v[r, sl] * cfb
            return 0

        lax.fori_loop(0, K // 16, group, 0)
        pltpu.sync_copy(rows_v, acc.at[dst_v], add=True)
        return 0

    lax.fori_loop(0, NCHUNK, chunk, 0)
    plsc.subcore_barrier()

    for t in range(RPS // ZR):
        rr = r0 + t * ZR
        pltpu.sync_copy(acc.at[pl.ds(rr, ZR), :], out.at[c, pl.ds(rr, ZR), :])


_sc_agg = pl.kernel(
    _sc_agg_body,
    out_type=jax.ShapeDtypeStruct((NC, NP, H), jnp.float32),
    mesh=plsc.VectorSubcoreMesh(core_axis_name="c", subcore_axis_name="s"),
    scratch_types=[
        pltpu.VMEM((K,), jnp.int32),
        pltpu.VMEM((K,), jnp.int32),
        pltpu.VMEM((K,), jnp.float32),
        pltpu.VMEM((K, H), jnp.float32),
        pltpu.VMEM((ZR, H), jnp.float32),
        pltpu.SemaphoreType.DMA,
        pltpu.VMEM_SHARED((NP, H), jnp.float32),
    ],
)


def _coeff_body(ea, nm, out):
    out[...] = ea[...] * nm[...]


def _coeff(ea_flat, nm_flat):
    ea2 = ea_flat.reshape(E // H, H)
    nm2 = nm_flat.reshape(E // H, H)
    out = pl.pallas_call(
        _coeff_body,
        out_shape=jax.ShapeDtypeStruct((E // H, H), jnp.float32),
    )(ea2, nm2)
    return out.reshape(E)


def _enc_body(x, w1, b1, w2, b2, out):
    h = jnp.maximum(x[...] * w1[...] + b1[...], 0.0)
    out[...] = jnp.dot(h, w2[...], preferred_element_type=jnp.float32) + b2[...]


def _enc(x, w1, b1, w2, b2):
    return pl.pallas_call(
        _enc_body,
        out_shape=jax.ShapeDtypeStruct((N, H), jnp.float32),
    )(x.reshape(N, 1), w1.reshape(1, H), b1.reshape(1, H), w2,
      b2.reshape(1, H))


def _mlp_body(parts, prev, w1, b1, w2, b2, out):
    x = parts[0, :N, :] + parts[1, :N, :]
    h = (jnp.dot(x, w1[:H, :], preferred_element_type=jnp.float32)
         + jnp.dot(prev[...], w1[H:, :], preferred_element_type=jnp.float32)
         + b1[...])
    h = jnp.maximum(h, 0.0)
    out[...] = jnp.dot(h, w2[...], preferred_element_type=jnp.float32) + b2[...]


def _mlp(parts, prev, w1, b1, w2, b2):
    return pl.pallas_call(
        _mlp_body,
        out_shape=jax.ShapeDtypeStruct((N, H), jnp.float32),
    )(parts, prev, w1, b1.reshape(1, H), w2, b2.reshape(1, H))


def _final_body(vals, cons, bv, bc, fw, fb, out):
    gids = lax.broadcasted_iota(jnp.int32, (1, NG), 1)
    ones = jnp.ones((N, 1), jnp.float32)

    def gmp(x, batch):
        oh = (batch == gids).astype(jnp.float32)
        ssum = lax.dot_general(oh, x, (((0,), (0,)), ((), ())),
                               preferred_element_type=jnp.float32)
        cnt = lax.dot_general(oh, ones, (((0,), (0,)), ((), ())),
                              preferred_element_type=jnp.float32)
        return ssum / jnp.maximum(cnt, 1.0)

    pred = gmp(vals[...], bv[...]) + gmp(cons[...], bc[...])
    out[...] = jnp.dot(pred, fw[...], preferred_element_type=jnp.float32) + fb[...]


def _final(vals, cons, bv, bc, fw, fb):
    return pl.pallas_call(
        _final_body,
        out_shape=jax.ShapeDtypeStruct((NG, H), jnp.float32),
    )(vals, cons, bv.reshape(N, 1), bc.reshape(N, 1), fw, fb.reshape(1, H))


def kernel(b, q, edge_index, edge_attr, norm_v2c, norm_c2v, batch_vals,
           batch_cons, num_graphs, be_W1, be_b1, be_W2, be_b2, qe_W1, qe_b1,
           qe_W2, qe_b2,
           conv0_v2c_W1, conv0_v2c_b1, conv0_v2c_W2, conv0_v2c_b2,
           conv0_c2v_W1, conv0_c2v_b1, conv0_c2v_W2, conv0_c2v_b2,
           conv1_v2c_W1, conv1_v2c_b1, conv1_v2c_W2, conv1_v2c_b2,
           conv1_c2v_W1, conv1_c2v_b1, conv1_c2v_W2, conv1_c2v_b2,
           conv2_v2c_W1, conv2_v2c_b1, conv2_v2c_W2, conv2_v2c_b2,
           conv2_c2v_W1, conv2_c2v_b1, conv2_c2v_W2, conv2_c2v_b2,
           fc_W, fc_b):
    src = edge_index[0].astype(jnp.int32)
    dst = edge_index[1].astype(jnp.int32)
    cf_v2c = _coeff(edge_attr.reshape(E), norm_v2c)
    cf_c2v = _coeff(edge_attr.reshape(E), norm_c2v)

    cons = _enc(b, be_W1, be_b1, be_W2, be_b2)
    vals = _enc(q, qe_W1, qe_b1, qe_W2, qe_b2)

    conv_w = (
        (conv0_v2c_W1, conv0_v2c_b1, conv0_v2c_W2, conv0_v2c_b2,
         conv0_c2v_W1, conv0_c2v_b1, conv0_c2v_W2, conv0_c2v_b2),
        (conv1_v2c_W1, conv1_v2c_b1, conv1_v2c_W2, conv1_v2c_b2,
         conv1_c2v_W1, conv1_c2v_b1, conv1_c2v_W2, conv1_c2v_b2),
        (conv2_v2c_W1, conv2_v2c_b1, conv2_v2c_W2, conv2_v2c_b2,
         conv2_c2v_W1, conv2_c2v_b1, conv2_c2v_W2, conv2_c2v_b2),
    )
    for (w1a, b1a, w2a, b2a, w1b, b1b, w2b, b2b) in conv_w:
        parts = _sc_agg(vals, src, dst, cf_v2c)
        cons = _mlp(parts, cons, w1a, b1a, w2a, b2a)
        parts = _sc_agg(cons, dst, src, cf_c2v)
        vals = _mlp(parts, vals, w1b, b1b, w2b, b2b)

    return _final(vals, cons, batch_vals.astype(jnp.int32),
                  batch_cons.astype(jnp.int32), fc_W, fc_b)
